# baseline (speedup 1.0000x reference)
import functools

import jax
import jax.numpy as jnp
from jax import lax
from jax.experimental import pallas as pl
from jax.experimental.pallas import tpu as pltpu
from jax.experimental.pallas import tpu_sc as plsc

_H = 224
_W = 224
_C = 96
_CO = 2 * _C
_RB = 32
_RCH = 56


def _tr_in_body(x_ref, o_ref):
    cols = [x_ref[0, r, :, :].T for r in range(_RB)]
    o_ref[0] = jnp.stack(cols, axis=1)


def _tr_out_body(x_ref, o_ref):
    rows = [x_ref[0, :, r, :].T for r in range(_RB)]
    o_ref[0] = jnp.stack(rows, axis=0)


def _conv_cm_body(xm1_ref, x0_ref, xp1_ref, wt_ref, b_ref, o_ref):
    i = pl.program_id(1)
    n_i = _H // _RB
    top = jnp.where(i > 0, xm1_ref[0][:, _RB - 1:, :], 0.0)
    bot = jnp.where(i < n_i - 1, xp1_ref[0][:, :1, :], 0.0)
    y = jnp.concatenate([top, x0_ref[0], bot], axis=1)
    zero = jnp.zeros((_CO, 1), jnp.float32)
    accs = []
    for r in range(_RB):
        zr = jnp.concatenate(
            [y[:, r, :], y[:, r + 1, :], y[:, r + 2, :]], axis=0)
        r0 = jnp.dot(wt_ref[0], zr, preferred_element_type=jnp.float32)
        r1 = jnp.dot(wt_ref[1], zr, preferred_element_type=jnp.float32)
        r2 = jnp.dot(wt_ref[2], zr, preferred_element_type=jnp.float32)
        acc = (jnp.concatenate([zero, r0[:, : _W - 1]], axis=1)
               + r1
               + jnp.concatenate([r2[:, 1:], zero], axis=1)
               + b_ref[...])
        accs.append(acc)
    o_ref[0] = jnp.stack(accs, axis=1)


def _tc_stage(inputs, wt, bias):
    B = inputs.shape[0]
    n_i = _H // _RB
    x_t = pl.pallas_call(
        _tr_in_body,
        grid=(B, n_i),
        in_specs=[pl.BlockSpec((1, _RB, _W, _C), lambda b, i: (b, i, 0, 0))],
        out_specs=pl.BlockSpec((1, _C, _RB, _W), lambda b, i: (b, 0, i, 0)),
        out_shape=jax.ShapeDtypeStruct((B, _C, _H, _W), jnp.float32),
    )(inputs)
    off_cm = pl.pallas_call(
        _conv_cm_body,
        grid=(B, n_i),
        in_specs=[
            pl.BlockSpec((1, _C, _RB, _W),
                         lambda b, i: (b, 0, jnp.maximum(i - 1, 0), 0)),
            pl.BlockSpec((1, _C, _RB, _W), lambda b, i: (b, 0, i, 0)),
            pl.BlockSpec((1, _C, _RB, _W),
                         lambda b, i: (b, 0, jnp.minimum(i + 1, n_i - 1), 0)),
            pl.BlockSpec((3, _CO, 3 * _C), lambda b, i: (0, 0, 0)),
            pl.BlockSpec((_CO, 1), lambda b, i: (0, 0)),
        ],
        out_specs=pl.BlockSpec((1, _CO, _RB, _W), lambda b, i: (b, 0, i, 0)),
        out_shape=jax.ShapeDtypeStruct((B, _CO, _H, _W), jnp.float32),
    )(x_t, x_t, x_t, wt, bias)
    return x_t, off_cm


def _tr_out(out_t, B):
    n_i = _H // _RB
    return pl.pallas_call(
        _tr_out_body,
        grid=(B, n_i),
        in_specs=[pl.BlockSpec((1, _C, _RB, _W), lambda b, i: (b, 0, i, 0))],
        out_specs=pl.BlockSpec((1, _RB, _W, _C), lambda b, i: (b, i, 0, 0)),
        out_shape=jax.ShapeDtypeStruct((B, _H, _W, _C), jnp.float32),
    )(out_t.reshape(B, _C, _H, _W))


def _sample(x_t, off_cm):
    B = x_t.shape[0]
    BC = B * _C
    hh2 = _H // 2
    info = plsc.get_sparse_core_info()
    nw = info.num_cores * info.num_subcores
    spw = BC // nw
    mesh = plsc.VectorSubcoreMesh(core_axis_name="c", subcore_axis_name="s")

    n_chunks = _H // _RCH
    per_half = hh2 // _RCH

    @functools.partial(
        pl.kernel, mesh=mesh,
        compiler_params=pltpu.CompilerParams(
            use_tc_tiling_on_sc=False, needs_layout_passes=False),
        out_type=jax.ShapeDtypeStruct((B, _C, _H, _W), jnp.float32),
        scratch_types=[
            pltpu.VMEM((_H, _W), jnp.float32),
            pltpu.VMEM((2 * _RCH, _W), jnp.float32),
            pltpu.VMEM((2 * _RCH, _W), jnp.float32),
            pltpu.VMEM((_RCH, _W), jnp.float32),
            pltpu.VMEM((_RCH, _W), jnp.float32),
            pltpu.SemaphoreType.DMA,
            pltpu.SemaphoreType.DMA,
            pltpu.SemaphoreType.DMA,
            pltpu.SemaphoreType.DMA,
        ],
    )
    def k(x_hbm, off_hbm, out_hbm, img_v,
          off_v0, off_v1, out_v0, out_v1,
          off_s0, off_s1, out_s0, out_s1):
        wid = lax.axis_index("s") * info.num_cores + lax.axis_index("c")
        iota = lax.iota(jnp.int32, 16)
        two_iota = iota * 2
        lanes_f = iota.astype(jnp.float32)
        xbase = [lanes_f + float(g * 16) for g in range(_W // 16)]
        colbase = [two_iota + 2 * (g * 16 - hh2 * (g // 7))
                   for g in range(_W // 16)]
        off_bufs = (off_v0, off_v1)
        out_bufs = (out_v0, out_v1)
        off_sems = (off_s0, off_s1)
        out_sems = (out_s0, out_s1)

        def off_src(b, c, t):
            p, r0 = t // per_half, (t % per_half) * _RCH
            return off_hbm.at[b, 2 * c + p, pl.ds(2 * r0, 2 * _RCH)]

        def out_dst(b, c, t):
            p, r0 = t // per_half, (t % per_half) * _RCH
            return out_hbm.at[b, c, pl.ds(p * hh2 + r0, _RCH)]

        def compute_chunk(t, off_v, out_v):
            p, r0 = t // per_half, (t % per_half) * _RCH

            @plsc.parallel_loop(0, _RCH)
            def row_body(hh):
                hf = (p * hh2 + r0 + hh).astype(jnp.float32)
                for g in range(_W // 16):
                    kk = g // 7
                    srow = jnp.full((16,), 2 * hh + kk, jnp.int32)
                    dyv = plsc.load_gather(off_v, [srow, colbase[g]])
                    dxv = plsc.load_gather(off_v, [srow, colbase[g] + 1])
                    yc = jnp.clip(hf + dyv, 0.0, float(_H - 1))
                    xc = jnp.clip(xbase[g] + dxv, 0.0, float(_W - 1))
                    y0 = yc.astype(jnp.int32)
                    x0 = xc.astype(jnp.int32)
                    fy = yc - y0.astype(jnp.float32)
                    fx = xc - x0.astype(jnp.float32)
                    y1 = jnp.minimum(y0 + 1, _H - 1)
                    x1 = jnp.minimum(x0 + 1, _W - 1)
                    v_lt = plsc.load_gather(img_v, [y0, x0])
                    v_rt = plsc.load_gather(img_v, [y1, x0])
                    v_lb = plsc.load_gather(img_v, [y0, x1])
                    v_rb = plsc.load_gather(img_v, [y1, x1])
                    vt = v_lt + (v_rt - v_lt) * fy
                    vb = v_lb + (v_rb - v_lb) * fy
                    out_v[hh, pl.ds(g * 16, 16)] = vt + (vb - vt) * fx

        def slab_body(s, carry):
            bc = wid * spw + s
            b = bc // _C
            c = bc % _C
            pltpu.async_copy(off_src(b, c, 0), off_bufs[0], off_sems[0])
            pltpu.sync_copy(x_hbm.at[b, c], img_v)
            for t in range(n_chunks):
                bb = t % 2
                pltpu.make_async_copy(
                    off_src(b, c, t), off_bufs[bb], off_sems[bb]).wait()
                if t + 1 < n_chunks:
                    pltpu.async_copy(
                        off_src(b, c, t + 1), off_bufs[1 - bb],
                        off_sems[1 - bb])
                if t >= 2:
                    pltpu.make_async_copy(
                        out_bufs[bb], out_dst(b, c, t - 2), out_sems[bb]).wait()
                compute_chunk(t, off_bufs[bb], out_bufs[bb])
                pltpu.async_copy(out_bufs[bb], out_dst(b, c, t), out_sems[bb])
            for t in range(n_chunks - 2, n_chunks):
                bb = t % 2
                pltpu.make_async_copy(
                    out_bufs[bb], out_dst(b, c, t), out_sems[bb]).wait()
            return carry

        lax.fori_loop(0, spw, slab_body, 0)

    return k(x_t, off_cm)


def kernel(inputs, W_offset, b_offset):
    B, H, Wd, C = inputs.shape
    wt = jnp.transpose(W_offset, (1, 3, 0, 2)).reshape(3, 2 * C, 3 * C)
    bias = b_offset.reshape(2 * C, 1)
    x_t, off_cm = _tc_stage(inputs, wt, bias)
    out_t = _sample(x_t, off_cm)
    return _tr_out(out_t, B)


# speedup vs baseline: 1.0407x; 1.0407x over previous
import functools

import jax
import jax.numpy as jnp
from jax import lax
from jax.experimental import pallas as pl
from jax.experimental.pallas import tpu as pltpu
from jax.experimental.pallas import tpu_sc as plsc

_H = 224
_W = 224
_C = 96
_CO = 2 * _C
_RB = 32
_RCH = 56


def _tr_in_body(x_ref, o_ref):
    cols = [x_ref[0, r, :, :].T for r in range(_RB)]
    o_ref[0] = jnp.stack(cols, axis=1)


def _tr_out_body(x_ref, o_ref):
    rows = [x_ref[0, :, r, :].T for r in range(_RB)]
    o_ref[0] = jnp.stack(rows, axis=0)


def _conv_cm_body(xm1_ref, x0_ref, xp1_ref, wt_ref, b_ref, o_ref):
    i = pl.program_id(1)
    n_i = _H // _RB
    top = jnp.where(i > 0, xm1_ref[0][:, _RB - 1:, :], 0.0)
    bot = jnp.where(i < n_i - 1, xp1_ref[0][:, :1, :], 0.0)
    y = jnp.concatenate([top, x0_ref[0], bot], axis=1)
    zero = jnp.zeros((_CO, 1), jnp.float32)
    accs = []
    for r in range(_RB):
        zr = jnp.concatenate(
            [y[:, r, :], y[:, r + 1, :], y[:, r + 2, :]], axis=0)
        r0 = jnp.dot(wt_ref[0], zr, preferred_element_type=jnp.float32)
        r1 = jnp.dot(wt_ref[1], zr, preferred_element_type=jnp.float32)
        r2 = jnp.dot(wt_ref[2], zr, preferred_element_type=jnp.float32)
        acc = (jnp.concatenate([zero, r0[:, : _W - 1]], axis=1)
               + r1
               + jnp.concatenate([r2[:, 1:], zero], axis=1)
               + b_ref[...])
        accs.append(acc)
    o_ref[0] = jnp.stack(accs, axis=1)


def _tc_stage(inputs, wt, bias):
    B = inputs.shape[0]
    n_i = _H // _RB
    x_t = pl.pallas_call(
        _tr_in_body,
        grid=(B, n_i),
        in_specs=[pl.BlockSpec((1, _RB, _W, _C), lambda b, i: (b, i, 0, 0))],
        out_specs=pl.BlockSpec((1, _C, _RB, _W), lambda b, i: (b, 0, i, 0)),
        out_shape=jax.ShapeDtypeStruct((B, _C, _H, _W), jnp.float32),
    )(inputs)
    off_cm = pl.pallas_call(
        _conv_cm_body,
        grid=(B, n_i),
        in_specs=[
            pl.BlockSpec((1, _C, _RB, _W),
                         lambda b, i: (b, 0, jnp.maximum(i - 1, 0), 0)),
            pl.BlockSpec((1, _C, _RB, _W), lambda b, i: (b, 0, i, 0)),
            pl.BlockSpec((1, _C, _RB, _W),
                         lambda b, i: (b, 0, jnp.minimum(i + 1, n_i - 1), 0)),
            pl.BlockSpec((3, _CO, 3 * _C), lambda b, i: (0, 0, 0)),
            pl.BlockSpec((_CO, 1), lambda b, i: (0, 0)),
        ],
        out_specs=pl.BlockSpec((1, _CO, _RB, _W), lambda b, i: (b, 0, i, 0)),
        out_shape=jax.ShapeDtypeStruct((B, _CO, _H, _W), jnp.float32),
    )(x_t, x_t, x_t, wt, bias)
    return x_t, off_cm


def _tr_out(out_t, B):
    n_i = _H // _RB
    return pl.pallas_call(
        _tr_out_body,
        grid=(B, n_i),
        in_specs=[pl.BlockSpec((1, _C, _RB, _W), lambda b, i: (b, 0, i, 0))],
        out_specs=pl.BlockSpec((1, _RB, _W, _C), lambda b, i: (b, i, 0, 0)),
        out_shape=jax.ShapeDtypeStruct((B, _H, _W, _C), jnp.float32),
    )(out_t.reshape(B, _C, _H, _W))


def _sample(x_t, off_cm):
    B = x_t.shape[0]
    BC = B * _C
    hh2 = _H // 2
    info = plsc.get_sparse_core_info()
    nw = info.num_cores * info.num_subcores
    spw = BC // nw
    mesh = plsc.VectorSubcoreMesh(core_axis_name="c", subcore_axis_name="s")

    n_chunks = _H // _RCH
    per_half = hh2 // _RCH

    hw = _H * _W

    @functools.partial(
        pl.kernel, mesh=mesh,
        compiler_params=pltpu.CompilerParams(
            use_tc_tiling_on_sc=False, needs_layout_passes=False),
        out_type=jax.ShapeDtypeStruct((BC, hw), jnp.float32),
        scratch_types=[
            pltpu.VMEM((hw,), jnp.float32),
            pltpu.VMEM((2 * _RCH * _W,), jnp.float32),
            pltpu.VMEM((2 * _RCH * _W,), jnp.float32),
            pltpu.VMEM((_RCH * _W,), jnp.float32),
            pltpu.VMEM((_RCH * _W,), jnp.float32),
            pltpu.SemaphoreType.DMA,
            pltpu.SemaphoreType.DMA,
            pltpu.SemaphoreType.DMA,
            pltpu.SemaphoreType.DMA,
        ],
    )
    def k(x_hbm, off_hbm, out_hbm, img_v,
          off_v0, off_v1, out_v0, out_v1,
          off_s0, off_s1, out_s0, out_s1):
        wid = lax.axis_index("s") * info.num_cores + lax.axis_index("c")
        iota = lax.iota(jnp.int32, 16)
        two_iota = iota * 2
        lanes_f = iota.astype(jnp.float32)
        xbase = [lanes_f + float(g * 16) for g in range(_W // 16)]
        colbase = [two_iota + 2 * (g * 16 - hh2 * (g // 7))
                   for g in range(_W // 16)]
        off_bufs = (off_v0, off_v1)
        out_bufs = (out_v0, out_v1)
        off_sems = (off_s0, off_s1)
        out_sems = (out_s0, out_s1)

        def off_src(bc, t):
            p, r0 = t // per_half, (t % per_half) * _RCH
            return off_hbm.at[bc, pl.ds(p * hw + 2 * r0 * _W, 2 * _RCH * _W)]

        def out_dst(bc, t):
            p, r0 = t // per_half, (t % per_half) * _RCH
            return out_hbm.at[bc, pl.ds((p * hh2 + r0) * _W, _RCH * _W)]

        def compute_chunk(t, off_v, out_v):
            p, r0 = t // per_half, (t % per_half) * _RCH

            @plsc.parallel_loop(0, _RCH)
            def row_body(hh):
                hf = (p * hh2 + r0 + hh).astype(jnp.float32)
                for g in range(_W // 16):
                    kk = g // 7
                    idx0 = (2 * hh + kk) * _W + colbase[g]
                    dyv = plsc.load_gather(off_v, [idx0])
                    dxv = plsc.load_gather(off_v, [idx0 + 1])
                    yc = jnp.clip(hf + dyv, 0.0, float(_H - 1))
                    xc = jnp.clip(xbase[g] + dxv, 0.0, float(_W - 1))
                    y0 = yc.astype(jnp.int32)
                    x0 = xc.astype(jnp.int32)
                    fy = yc - y0.astype(jnp.float32)
                    fx = xc - x0.astype(jnp.float32)
                    y1 = jnp.minimum(y0 + 1, _H - 1)
                    x1 = jnp.minimum(x0 + 1, _W - 1)
                    r0i = y0 * _W
                    r1i = y1 * _W
                    v_lt = plsc.load_gather(img_v, [r0i + x0])
                    v_rt = plsc.load_gather(img_v, [r1i + x0])
                    v_lb = plsc.load_gather(img_v, [r0i + x1])
                    v_rb = plsc.load_gather(img_v, [r1i + x1])
                    vt = v_lt + (v_rt - v_lt) * fy
                    vb = v_lb + (v_rb - v_lb) * fy
                    out_v[pl.ds(hh * _W + g * 16, 16)] = vt + (vb - vt) * fx

        def slab_body(s, carry):
            bc = wid * spw + s
            pltpu.async_copy(off_src(bc, 0), off_bufs[0], off_sems[0])
            pltpu.sync_copy(x_hbm.at[bc], img_v)
            for t in range(n_chunks):
                bb = t % 2
                pltpu.make_async_copy(
                    off_src(bc, t), off_bufs[bb], off_sems[bb]).wait()
                if t + 1 < n_chunks:
                    pltpu.async_copy(
                        off_src(bc, t + 1), off_bufs[1 - bb],
                        off_sems[1 - bb])
                if t >= 2:
                    pltpu.make_async_copy(
                        out_bufs[bb], out_dst(bc, t - 2), out_sems[bb]).wait()
                compute_chunk(t, off_bufs[bb], out_bufs[bb])
                pltpu.async_copy(out_bufs[bb], out_dst(bc, t), out_sems[bb])
            for t in range(n_chunks - 2, n_chunks):
                bb = t % 2
                pltpu.make_async_copy(
                    out_bufs[bb], out_dst(bc, t), out_sems[bb]).wait()
            return carry

        lax.fori_loop(0, spw, slab_body, 0)

    return k(x_t.reshape(BC, hw), off_cm.reshape(BC, 2 * hw))


def kernel(inputs, W_offset, b_offset):
    B, H, Wd, C = inputs.shape
    wt = jnp.transpose(W_offset, (1, 3, 0, 2)).reshape(3, 2 * C, 3 * C)
    bias = b_offset.reshape(2 * C, 1)
    x_t, off_cm = _tc_stage(inputs, wt, bias)
    out_t = _sample(x_t, off_cm)
    return _tr_out(out_t, B)


# speedup vs baseline: 1.0584x; 1.0169x over previous
import functools

import jax
import jax.numpy as jnp
from jax import lax
from jax.experimental import pallas as pl
from jax.experimental.pallas import tpu as pltpu
from jax.experimental.pallas import tpu_sc as plsc

_H = 224
_W = 224
_C = 96
_CO = 2 * _C
_RB = 32
_RCH = 56


def _tr_in_body(x_ref, o_ref):
    cols = [x_ref[0, r, :, :].T for r in range(_RB)]
    o_ref[0] = jnp.stack(cols, axis=1)


def _tr_out_body(x_ref, o_ref):
    rows = [x_ref[0, :, r, :].T for r in range(_RB)]
    o_ref[0] = jnp.stack(rows, axis=0)


def _conv_cm_body(xm1_ref, x0_ref, xp1_ref, wt_ref, b_ref, o_ref):
    i = pl.program_id(1)
    n_i = _H // _RB
    top = jnp.where(i > 0, xm1_ref[0][:, _RB - 1:, :], 0.0)
    bot = jnp.where(i < n_i - 1, xp1_ref[0][:, :1, :], 0.0)
    y = jnp.concatenate([top, x0_ref[0], bot], axis=1)
    zero = jnp.zeros((_CO, 1), jnp.float32)
    accs = []
    for r in range(_RB):
        zr = jnp.concatenate(
            [y[:, r, :], y[:, r + 1, :], y[:, r + 2, :]], axis=0)
        r0 = jnp.dot(wt_ref[0], zr, preferred_element_type=jnp.float32)
        r1 = jnp.dot(wt_ref[1], zr, preferred_element_type=jnp.float32)
        r2 = jnp.dot(wt_ref[2], zr, preferred_element_type=jnp.float32)
        acc = (jnp.concatenate([zero, r0[:, : _W - 1]], axis=1)
               + r1
               + jnp.concatenate([r2[:, 1:], zero], axis=1)
               + b_ref[...])
        accs.append(acc)
    o_ref[0] = jnp.stack(accs, axis=1)


def _tc_stage(inputs, wt, bias):
    B = inputs.shape[0]
    n_i = _H // _RB
    x_t = pl.pallas_call(
        _tr_in_body,
        grid=(B, n_i),
        in_specs=[pl.BlockSpec((1, _RB, _W, _C), lambda b, i: (b, i, 0, 0))],
        out_specs=pl.BlockSpec((1, _C, _RB, _W), lambda b, i: (b, 0, i, 0)),
        out_shape=jax.ShapeDtypeStruct((B, _C, _H, _W), jnp.float32),
    )(inputs)
    off_cm = pl.pallas_call(
        _conv_cm_body,
        grid=(B, n_i),
        in_specs=[
            pl.BlockSpec((1, _C, _RB, _W),
                         lambda b, i: (b, 0, jnp.maximum(i - 1, 0), 0)),
            pl.BlockSpec((1, _C, _RB, _W), lambda b, i: (b, 0, i, 0)),
            pl.BlockSpec((1, _C, _RB, _W),
                         lambda b, i: (b, 0, jnp.minimum(i + 1, n_i - 1), 0)),
            pl.BlockSpec((3, _CO, 3 * _C), lambda b, i: (0, 0, 0)),
            pl.BlockSpec((_CO, 1), lambda b, i: (0, 0)),
        ],
        out_specs=pl.BlockSpec((1, _CO, _RB, _W), lambda b, i: (b, 0, i, 0)),
        out_shape=jax.ShapeDtypeStruct((B, _CO, _H, _W), jnp.float32),
    )(x_t, x_t, x_t, wt, bias)
    return x_t, off_cm


def _tr_out(out_t, B):
    n_i = _H // _RB
    return pl.pallas_call(
        _tr_out_body,
        grid=(B, n_i),
        in_specs=[pl.BlockSpec((1, _C, _RB, _W), lambda b, i: (b, 0, i, 0))],
        out_specs=pl.BlockSpec((1, _RB, _W, _C), lambda b, i: (b, i, 0, 0)),
        out_shape=jax.ShapeDtypeStruct((B, _H, _W, _C), jnp.float32),
    )(out_t.reshape(B, _C, _H, _W))


def _sample(x_t, off_cm):
    B = x_t.shape[0]
    BC = B * _C
    hh2 = _H // 2
    info = plsc.get_sparse_core_info()
    nw = info.num_cores * info.num_subcores
    spw = BC // nw
    mesh = plsc.VectorSubcoreMesh(core_axis_name="c", subcore_axis_name="s")

    n_chunks = _H // _RCH
    per_half = hh2 // _RCH

    hw = _H * _W

    @functools.partial(
        pl.kernel, mesh=mesh,
        compiler_params=pltpu.CompilerParams(
            use_tc_tiling_on_sc=False, needs_layout_passes=False),
        out_type=jax.ShapeDtypeStruct((BC, hw), jnp.float32),
        scratch_types=[
            pltpu.VMEM((hw,), jnp.float32),
            pltpu.VMEM((2 * _RCH * _W,), jnp.float32),
            pltpu.VMEM((2 * _RCH * _W,), jnp.float32),
            pltpu.VMEM((_RCH * _W,), jnp.float32),
            pltpu.VMEM((_RCH * _W,), jnp.float32),
            pltpu.SemaphoreType.DMA,
            pltpu.SemaphoreType.DMA,
            pltpu.SemaphoreType.DMA,
            pltpu.SemaphoreType.DMA,
        ],
    )
    def k(x_hbm, off_hbm, out_hbm, img_v,
          off_v0, off_v1, out_v0, out_v1,
          off_s0, off_s1, out_s0, out_s1):
        wid = lax.axis_index("s") * info.num_cores + lax.axis_index("c")
        iota = lax.iota(jnp.int32, 16)
        two_iota = iota * 2
        lanes_f = iota.astype(jnp.float32)
        xbase = [lanes_f + float(g * 16) for g in range(_W // 16)]
        off_bufs = (off_v0, off_v1)
        out_bufs = (out_v0, out_v1)
        off_sems = (off_s0, off_s1)
        out_sems = (out_s0, out_s1)

        def off_src(bc, t):
            p, r0 = t // per_half, (t % per_half) * _RCH
            return off_hbm.at[bc, pl.ds(p * hw + 2 * r0 * _W, 2 * _RCH * _W)]

        def out_dst(bc, t):
            p, r0 = t // per_half, (t % per_half) * _RCH
            return out_hbm.at[bc, pl.ds((p * hh2 + r0) * _W, _RCH * _W)]

        def compute_chunk(t, off_v, out_v):
            p, r0 = t // per_half, (t % per_half) * _RCH

            @plsc.parallel_loop(0, _RCH)
            def row_body(hh):
                hf = (p * hh2 + r0 + hh).astype(jnp.float32)
                for g in range(_W // 16):
                    kk = g // 7
                    base = (2 * hh + kk) * _W + 2 * (g * 16 - hh2 * kk)
                    idx0 = base + two_iota
                    dyv = plsc.load_gather(off_v, [idx0])
                    dxv = plsc.load_gather(off_v, [idx0 + 1])
                    yc = jnp.clip(hf + dyv, 0.0, float(_H - 1))
                    xc = jnp.clip(xbase[g] + dxv, 0.0, float(_W - 1))
                    y0 = yc.astype(jnp.int32)
                    x0 = xc.astype(jnp.int32)
                    fy = yc - y0.astype(jnp.float32)
                    fx = xc - x0.astype(jnp.float32)
                    y1 = jnp.minimum(y0 + 1, _H - 1)
                    x1 = jnp.minimum(x0 + 1, _W - 1)
                    r0i = y0 * _W
                    r1i = y1 * _W
                    v_lt = plsc.load_gather(img_v, [r0i + x0])
                    v_rt = plsc.load_gather(img_v, [r1i + x0])
                    v_lb = plsc.load_gather(img_v, [r0i + x1])
                    v_rb = plsc.load_gather(img_v, [r1i + x1])
                    vt = v_lt + (v_rt - v_lt) * fy
                    vb = v_lb + (v_rb - v_lb) * fy
                    out_v[pl.ds(hh * _W + g * 16, 16)] = vt + (vb - vt) * fx

        def slab_body(s, carry):
            bc = wid * spw + s
            pltpu.async_copy(off_src(bc, 0), off_bufs[0], off_sems[0])
            pltpu.sync_copy(x_hbm.at[bc], img_v)
            for t in range(n_chunks):
                bb = t % 2
                pltpu.make_async_copy(
                    off_src(bc, t), off_bufs[bb], off_sems[bb]).wait()
                if t + 1 < n_chunks:
                    pltpu.async_copy(
                        off_src(bc, t + 1), off_bufs[1 - bb],
                        off_sems[1 - bb])
                if t >= 2:
                    pltpu.make_async_copy(
                        out_bufs[bb], out_dst(bc, t - 2), out_sems[bb]).wait()
                compute_chunk(t, off_bufs[bb], out_bufs[bb])
                pltpu.async_copy(out_bufs[bb], out_dst(bc, t), out_sems[bb])
            for t in range(n_chunks - 2, n_chunks):
                bb = t % 2
                pltpu.make_async_copy(
                    out_bufs[bb], out_dst(bc, t), out_sems[bb]).wait()
            return carry

        lax.fori_loop(0, spw, slab_body, 0)

    return k(x_t.reshape(BC, hw), off_cm.reshape(BC, 2 * hw))


def kernel(inputs, W_offset, b_offset):
    B, H, Wd, C = inputs.shape
    wt = jnp.transpose(W_offset, (1, 3, 0, 2)).reshape(3, 2 * C, 3 * C)
    bias = b_offset.reshape(2 * C, 1)
    x_t, off_cm = _tc_stage(inputs, wt, bias)
    out_t = _sample(x_t, off_cm)
    return _tr_out(out_t, B)
